# baseline (speedup 1.0000x reference)
import functools

import jax
import jax.numpy as jnp
from jax import lax
from jax.experimental import pallas as pl
from jax.experimental.pallas import tpu as pltpu
from jax.experimental.pallas import tpu_sc as plsc

B = 16384
F = 64
ROWS_PER_TILE = 8
N_TILES = 1000000 // ROWS_PER_TILE
NC = 2
NS = 16
NW = NC * NS
BPW = B // NW

_mesh = plsc.VectorSubcoreMesh(core_axis_name="c", subcore_axis_name="s")


@functools.partial(
    pl.kernel,
    mesh=_mesh,
    out_type=jax.ShapeDtypeStruct((NW, BPW), jnp.float32),
    scratch_types=[
        pltpu.VMEM((8, 64), jnp.int32),
        pltpu.VMEM((8, 64), jnp.int32),
        pltpu.VMEM((BPW // 2, 2 * F), jnp.float32),
        pltpu.VMEM((BPW // 2, 2 * F), jnp.float32),
        pltpu.VMEM((BPW,), jnp.float32),
        pltpu.SemaphoreType.DMA,
    ],
    compiler_params=pltpu.CompilerParams(needs_layout_passes=False),
)
def _cf_kernel(user_hbm, item_hbm, ut_hbm, it_hbm, out_hbm,
               uraw_v, iraw_v, urows_v, irows_v, out_v, sem):
    wid = lax.axis_index("s") * NC + lax.axis_index("c")

    pltpu.sync_copy(user_hbm.at[wid], uraw_v)
    pltpu.sync_copy(item_hbm.at[wid], iraw_v)

    def fire_body(g, carry):
        u16 = uraw_v[g >> 2, pl.ds((g & 3) * 16, 16)]
        i16 = iraw_v[g >> 2, pl.ds((g & 3) * 16, 16)]
        for l in range(16):
            uidx = u16[l]
            iidx = i16[l]
            dst_row = g * 8 + (l >> 1)
            dst_off = (l & 1) * F
            pltpu.async_copy(
                ut_hbm.at[uidx],
                urows_v.at[dst_row, pl.ds(dst_off, F)], sem)
            pltpu.async_copy(
                it_hbm.at[iidx],
                irows_v.at[dst_row, pl.ds(dst_off, F)], sem)
        return carry

    lax.fori_loop(0, BPW // 16, fire_body, 0)

    def drain_body(k, carry):
        pltpu.make_async_copy(
            ut_hbm.at[0], urows_v.at[0, pl.ds(0, F)], sem).wait()
        pltpu.make_async_copy(
            it_hbm.at[0], irows_v.at[0, pl.ds(0, F)], sem).wait()
        return carry

    lax.fori_loop(0, BPW, drain_body, 0)

    lane = lax.iota(jnp.int32, 16)
    half = lane >> 1
    colbase = (lane & 1) * F

    def group_body(g, carry):
        rvec = g * 8 + half
        acc = jnp.zeros((16,), jnp.float32)
        for c in range(F):
            cvec = colbase + c
            u = plsc.load_gather(urows_v, [rvec, cvec])
            iv = plsc.load_gather(irows_v, [rvec, cvec])
            acc = acc + u * iv
        out_v[pl.ds(g * 16, 16)] = 1.0 / (1.0 + jnp.exp(-acc))
        return carry

    lax.fori_loop(0, BPW // 16, group_body, 0)

    pltpu.sync_copy(out_v, out_hbm.at[wid])


def kernel(user, item, user_table, item_table):
    u2 = user.astype(jnp.int32).reshape(NW, 8, 64)
    i2 = item.astype(jnp.int32).reshape(NW, 8, 64)
    out = _cf_kernel(u2, i2, user_table, item_table)
    return out.reshape(B)


# speedup vs baseline: 1.3888x; 1.3888x over previous
import functools

import jax
import jax.numpy as jnp
from jax import lax
from jax.experimental import pallas as pl
from jax.experimental.pallas import tpu as pltpu
from jax.experimental.pallas import tpu_sc as plsc

B = 16384
F = 64
NV = 1000000
NC = 2
NS = 16
NW = NC * NS
BPW = B // NW
BLK = 128
NBUF = 2

_mesh = plsc.VectorSubcoreMesh(core_axis_name="c", subcore_axis_name="s")


@functools.partial(
    pl.kernel,
    mesh=_mesh,
    out_type=jax.ShapeDtypeStruct((NW, BPW), jnp.float32),
    scratch_types=[
        pltpu.VMEM((2, BPW), jnp.int32),
        pltpu.SMEM((BPW,), jnp.int32),
        pltpu.SMEM((BPW,), jnp.int32),
        pltpu.VMEM((NBUF, F, BLK), jnp.float32),
        pltpu.VMEM((NBUF, F, BLK), jnp.float32),
        pltpu.VMEM((F, BPW), jnp.float32),
        pltpu.VMEM((F, BPW), jnp.float32),
        pltpu.VMEM((BPW,), jnp.float32),
        pltpu.SemaphoreType.DMA,
        pltpu.SemaphoreType.DMA,
    ],
    compiler_params=pltpu.CompilerParams(needs_layout_passes=False),
)
def _cf_kernel(user_hbm, item_hbm, utT_hbm, itT_hbm, out_hbm,
               idxstage_v, uidx_s, iidx_s, ublk_v, iblk_v,
               urowsT_v, irowsT_v, out_v, sem0, sem1):
    wid = lax.axis_index("s") * NC + lax.axis_index("c")
    sems = [sem0, sem1]

    pltpu.sync_copy(user_hbm.at[wid], idxstage_v.at[0])
    pltpu.sync_copy(item_hbm.at[wid], idxstage_v.at[1])

    def smem_fill(g, carry):
        u16 = idxstage_v[0, pl.ds(g * 16, 16)]
        i16 = idxstage_v[1, pl.ds(g * 16, 16)]
        for l in range(16):
            uidx_s[g * 16 + l] = u16[l]
            iidx_s[g * 16 + l] = i16[l]
        return carry

    lax.fori_loop(0, BPW // 16, smem_fill, 0)

    lane = lax.iota(jnp.int32, 16)

    def fire(k, slot):
        u = uidx_s[k]
        iv = iidx_s[k]
        pltpu.async_copy(
            utT_hbm.at[:, pl.ds((u >> 7) * BLK, BLK)],
            ublk_v.at[slot], sems[slot])
        pltpu.async_copy(
            itT_hbm.at[:, pl.ds((iv >> 7) * BLK, BLK)],
            iblk_v.at[slot], sems[slot])

    def extract(k, slot):
        pltpu.make_async_copy(
            utT_hbm.at[:, pl.ds(0, BLK)], ublk_v.at[slot], sems[slot]).wait()
        pltpu.make_async_copy(
            itT_hbm.at[:, pl.ds(0, BLK)], iblk_v.at[slot], sems[slot]).wait()
        cu = jnp.full((16,), uidx_s[k] & 127, jnp.int32)
        ci = jnp.full((16,), iidx_s[k] & 127, jnp.int32)
        kk = jnp.full((16,), k, jnp.int32)
        for s in range(4):
            fvec = s * 16 + lane
            uval = plsc.load_gather(ublk_v.at[slot], [fvec, cu])
            ival = plsc.load_gather(iblk_v.at[slot], [fvec, ci])
            plsc.store_scatter(urowsT_v, [fvec, kk], uval)
            plsc.store_scatter(irowsT_v, [fvec, kk], ival)

    def ring_body(t, carry):
        for p in range(NBUF):
            k = t * NBUF + p
            extract(k, p)

            @pl.when(k + NBUF < BPW)
            def _():
                fire(k + NBUF, p)

        return carry

    for p in range(NBUF):
        fire(p, p)
    lax.fori_loop(0, BPW // NBUF, ring_body, 0)

    def group_body(g, carry):
        acc = jnp.zeros((16,), jnp.float32)
        for f in range(F):
            u = urowsT_v[f, pl.ds(g * 16, 16)]
            iv = irowsT_v[f, pl.ds(g * 16, 16)]
            acc = acc + u * iv
        out_v[pl.ds(g * 16, 16)] = 1.0 / (1.0 + jnp.exp(-acc))
        return carry

    lax.fori_loop(0, BPW // 16, group_body, 0)

    pltpu.sync_copy(out_v, out_hbm.at[wid])


def kernel(user, item, user_table, item_table):
    u2 = user.astype(jnp.int32).reshape(NW, BPW)
    i2 = item.astype(jnp.int32).reshape(NW, BPW)
    out = _cf_kernel(u2, i2, user_table.T, item_table.T)
    return out.reshape(B)


# speedup vs baseline: 1.7117x; 1.2325x over previous
import functools

import jax
import jax.numpy as jnp
from jax import lax
from jax.experimental import pallas as pl
from jax.experimental.pallas import tpu as pltpu
from jax.experimental.pallas import tpu_sc as plsc

B = 16384
F = 64
NV = 1000000
NC = 2
NS = 16
NW = NC * NS
BPW = B // NW
BLK = 128
NBUF = 4

_mesh = plsc.VectorSubcoreMesh(core_axis_name="c", subcore_axis_name="s")


@functools.partial(
    pl.kernel,
    mesh=_mesh,
    out_type=jax.ShapeDtypeStruct((NW, BPW), jnp.float32),
    scratch_types=[
        pltpu.VMEM((2, BPW), jnp.int32),
        pltpu.SMEM((BPW,), jnp.int32),
        pltpu.SMEM((BPW,), jnp.int32),
        pltpu.VMEM((NBUF, F, BLK), jnp.float32),
        pltpu.VMEM((NBUF, F, BLK), jnp.float32),
        pltpu.VMEM((F, BPW), jnp.float32),
        pltpu.VMEM((BPW,), jnp.float32),
        pltpu.SemaphoreType.DMA,
        pltpu.SemaphoreType.DMA,
        pltpu.SemaphoreType.DMA,
        pltpu.SemaphoreType.DMA,
    ],
    compiler_params=pltpu.CompilerParams(needs_layout_passes=False),
)
def _cf_kernel(user_hbm, item_hbm, utT_hbm, itT_hbm, out_hbm,
               idxstage_v, uidx_s, iidx_s, ublk_v, iblk_v,
               prodT_v, out_v, sem0, sem1, sem2, sem3):
    wid = lax.axis_index("s") * NC + lax.axis_index("c")
    sems = [sem0, sem1, sem2, sem3]

    pltpu.sync_copy(user_hbm.at[wid], idxstage_v.at[0])
    pltpu.sync_copy(item_hbm.at[wid], idxstage_v.at[1])

    def smem_fill(g, carry):
        u16 = idxstage_v[0, pl.ds(g * 16, 16)]
        i16 = idxstage_v[1, pl.ds(g * 16, 16)]
        for l in range(16):
            uidx_s[g * 16 + l] = u16[l]
            iidx_s[g * 16 + l] = i16[l]
        return carry

    lax.fori_loop(0, BPW // 16, smem_fill, 0)

    lane = lax.iota(jnp.int32, 16)

    def fire(k, slot):
        u = uidx_s[k]
        iv = iidx_s[k]
        pltpu.async_copy(
            utT_hbm.at[:, pl.ds((u >> 7) * BLK, BLK)],
            ublk_v.at[slot], sems[slot])
        pltpu.async_copy(
            itT_hbm.at[:, pl.ds((iv >> 7) * BLK, BLK)],
            iblk_v.at[slot], sems[slot])

    def extract(k, slot):
        pltpu.make_async_copy(
            utT_hbm.at[:, pl.ds(0, BLK)], ublk_v.at[slot], sems[slot]).wait()
        pltpu.make_async_copy(
            itT_hbm.at[:, pl.ds(0, BLK)], iblk_v.at[slot], sems[slot]).wait()
        cu = jnp.full((16,), uidx_s[k] & 127, jnp.int32)
        ci = jnp.full((16,), iidx_s[k] & 127, jnp.int32)
        kk = jnp.full((16,), k, jnp.int32)
        for s in range(4):
            fvec = s * 16 + lane
            uval = plsc.load_gather(ublk_v.at[slot], [fvec, cu])
            ival = plsc.load_gather(iblk_v.at[slot], [fvec, ci])
            plsc.store_scatter(prodT_v, [fvec, kk], uval * ival)

    def ring_body(t, carry):
        for p in range(NBUF):
            k = t * NBUF + p
            extract(k, p)

            @pl.when(k + NBUF < BPW)
            def _():
                fire(k + NBUF, p)

        return carry

    for p in range(NBUF):
        fire(p, p)
    lax.fori_loop(0, BPW // NBUF, ring_body, 0)

    def group_body(g, carry):
        acc = jnp.zeros((16,), jnp.float32)
        for f in range(F):
            acc = acc + prodT_v[f, pl.ds(g * 16, 16)]
        out_v[pl.ds(g * 16, 16)] = 1.0 / (1.0 + jnp.exp(-acc))
        return carry

    lax.fori_loop(0, BPW // 16, group_body, 0)

    pltpu.sync_copy(out_v, out_hbm.at[wid])


def kernel(user, item, user_table, item_table):
    u2 = user.astype(jnp.int32).reshape(NW, BPW)
    i2 = item.astype(jnp.int32).reshape(NW, BPW)
    out = _cf_kernel(u2, i2, user_table.T, item_table.T)
    return out.reshape(B)


# speedup vs baseline: 1.8926x; 1.1057x over previous
import functools

import jax
import jax.numpy as jnp
from jax import lax
from jax.experimental import pallas as pl
from jax.experimental.pallas import tpu as pltpu
from jax.experimental.pallas import tpu_sc as plsc

B = 16384
F = 64
NC = 2
NS = 16
NW = NC * NS
BPW = B // NW
BLK = 128
NBUF = 4
PF = NBUF - 1

_mesh = plsc.VectorSubcoreMesh(core_axis_name="c", subcore_axis_name="s")


@functools.partial(
    pl.kernel,
    mesh=_mesh,
    out_type=jax.ShapeDtypeStruct((NW, BPW), jnp.float32),
    scratch_types=[
        pltpu.VMEM((2, BPW), jnp.int32),
        pltpu.SMEM((BPW,), jnp.int32),
        pltpu.SMEM((BPW,), jnp.int32),
        pltpu.VMEM((NBUF, F, BLK), jnp.float32),
        pltpu.VMEM((NBUF, F, BLK), jnp.float32),
        pltpu.VMEM((F, BPW), jnp.float32),
        pltpu.VMEM((BPW,), jnp.float32),
        pltpu.SemaphoreType.DMA,
        pltpu.SemaphoreType.DMA,
        pltpu.SemaphoreType.DMA,
        pltpu.SemaphoreType.DMA,
        pltpu.SemaphoreType.DMA,
    ],
    compiler_params=pltpu.CompilerParams(needs_layout_passes=False),
)
def _cf_kernel(user_hbm, item_hbm, utT_hbm, itT_hbm, out_hbm,
               idxstage_v, uidx_s, iidx_s, ublk_v, iblk_v,
               prodT_v, out_v, usem, isem0, isem1, isem2, isem3):
    wid = lax.axis_index("s") * NC + lax.axis_index("c")
    isems = [isem0, isem1, isem2, isem3]

    pltpu.sync_copy(user_hbm.at[wid], idxstage_v.at[0])
    pltpu.sync_copy(item_hbm.at[wid], idxstage_v.at[1])

    def smem_fill(g, carry):
        u16 = idxstage_v[0, pl.ds(g * 16, 16)]
        i16 = idxstage_v[1, pl.ds(g * 16, 16)]
        for l in range(16):
            uidx_s[g * 16 + l] = u16[l]
            iidx_s[g * 16 + l] = i16[l]
        return carry

    lax.fori_loop(0, BPW // 16, smem_fill, 0)

    lane = lax.iota(jnp.int32, 16)

    def ublock_new(k):
        kc = jnp.minimum(k, BPW - 1)
        blk = uidx_s[kc] >> 7
        prev = uidx_s[jnp.maximum(kc - 1, 0)] >> 7
        return (kc == 0) | (blk != prev)

    def fire_user(k, slot):
        u = uidx_s[jnp.minimum(k, BPW - 1)]
        pltpu.async_copy(
            utT_hbm.at[:, pl.ds((u >> 7) * BLK, BLK)],
            ublk_v.at[slot], usem)

    def fire_item(k, p):
        iv = iidx_s[k]
        pltpu.async_copy(
            itT_hbm.at[:, pl.ds((iv >> 7) * BLK, BLK)],
            iblk_v.at[p], isems[p])

    nf = jnp.int32(0)
    for k0 in range(PF):
        cond = ublock_new(jnp.int32(k0))
        slot = nf & (NBUF - 1)
        pl.when(cond)(lambda k0=k0, slot=slot: fire_user(k0, slot))
        nf = nf + cond.astype(jnp.int32)
        fire_item(k0, k0)

    def ring_body(t, carry):
        nf, cus = carry
        for p in range(NBUF):
            k = t * NBUF + p
            cond_f = (k + PF < BPW) & ublock_new(k + PF)
            fslot = nf & (NBUF - 1)
            pl.when(cond_f)(
                lambda k=k, fslot=fslot: fire_user(k + PF, fslot))
            nf = nf + cond_f.astype(jnp.int32)
            pl.when(k + PF < BPW)(
                lambda k=k, p=p: fire_item(k + PF, (p + PF) % NBUF))

            cond_w = ublock_new(k)

            @pl.when(cond_w)
            def _():
                pltpu.make_async_copy(
                    utT_hbm.at[:, pl.ds(0, BLK)], ublk_v.at[0], usem).wait()

            cus = cus + cond_w.astype(jnp.int32)
            uslot = jnp.full((16,), (cus - 1) & (NBUF - 1), jnp.int32)
            pltpu.make_async_copy(
                itT_hbm.at[:, pl.ds(0, BLK)], iblk_v.at[p], isems[p]).wait()

            cu = jnp.full((16,), uidx_s[k] & 127, jnp.int32)
            ci = jnp.full((16,), iidx_s[k] & 127, jnp.int32)
            kk = jnp.full((16,), k, jnp.int32)
            for s in range(4):
                fvec = s * 16 + lane
                uval = plsc.load_gather(ublk_v, [uslot, fvec, cu])
                ival = plsc.load_gather(iblk_v.at[p], [fvec, ci])
                plsc.store_scatter(prodT_v, [fvec, kk], uval * ival)
        return nf, cus

    lax.fori_loop(0, BPW // NBUF, ring_body, (nf, jnp.int32(0)))

    def group_body(g, carry):
        acc = jnp.zeros((16,), jnp.float32)
        for f in range(F):
            acc = acc + prodT_v[f, pl.ds(g * 16, 16)]
        out_v[pl.ds(g * 16, 16)] = 1.0 / (1.0 + jnp.exp(-acc))
        return carry

    lax.fori_loop(0, BPW // 16, group_body, 0)

    pltpu.sync_copy(out_v, out_hbm.at[wid])


def kernel(user, item, user_table, item_table):
    perm = jnp.argsort(user)
    us = user[perm].astype(jnp.int32).reshape(NW, BPW)
    its = item[perm].astype(jnp.int32).reshape(NW, BPW)
    outs = _cf_kernel(us, its, user_table.T, item_table.T)
    return jnp.zeros((B,), jnp.float32).at[perm].set(outs.reshape(B))


# speedup vs baseline: 2.2020x; 1.1635x over previous
import functools

import jax
import jax.numpy as jnp
from jax import lax
from jax.experimental import pallas as pl
from jax.experimental.pallas import tpu as pltpu
from jax.experimental.pallas import tpu_sc as plsc

B = 16384
F = 64
NC = 2
NS = 16
NW = NC * NS
BPW = B // NW
BLK = 128
NBUF = 4
PF = NBUF - 1

_mesh = plsc.VectorSubcoreMesh(core_axis_name="c", subcore_axis_name="s")


@functools.partial(
    pl.kernel,
    mesh=_mesh,
    out_type=jax.ShapeDtypeStruct((NW, BPW), jnp.float32),
    scratch_types=[
        pltpu.VMEM((2, BPW), jnp.int32),
        pltpu.SMEM((BPW,), jnp.int32),
        pltpu.SMEM((BPW,), jnp.int32),
        pltpu.VMEM((NBUF, F, BLK), jnp.float32),
        pltpu.VMEM((NBUF, F, BLK), jnp.float32),
        pltpu.VMEM((F, BPW), jnp.float32),
        pltpu.VMEM((BPW,), jnp.float32),
        pltpu.SemaphoreType.DMA,
        pltpu.SemaphoreType.DMA,
        pltpu.SemaphoreType.DMA,
        pltpu.SemaphoreType.DMA,
        pltpu.SemaphoreType.DMA,
    ],
    compiler_params=pltpu.CompilerParams(needs_layout_passes=False),
)
def _cf_kernel(user_hbm, item_hbm, utT_hbm, itT_hbm, out_hbm,
               idxstage_v, uidx_s, iidx_s, ublk_v, iblk_v,
               prodT_v, out_v, usem, isem0, isem1, isem2, isem3):
    wid = lax.axis_index("s") * NC + lax.axis_index("c")
    isems = [isem0, isem1, isem2, isem3]

    pltpu.sync_copy(user_hbm.at[wid], idxstage_v.at[0])
    pltpu.sync_copy(item_hbm.at[wid], idxstage_v.at[1])

    def smem_fill(g, carry):
        u16 = idxstage_v[0, pl.ds(g * 16, 16)]
        i16 = idxstage_v[1, pl.ds(g * 16, 16)]
        for l in range(16):
            uidx_s[g * 16 + l] = u16[l]
            iidx_s[g * 16 + l] = i16[l]
        return carry

    lax.fori_loop(0, BPW // 16, smem_fill, 0)

    lane = lax.iota(jnp.int32, 16)

    def ublock_new(k):
        kc = jnp.minimum(k, BPW - 1)
        blk = uidx_s[kc] >> 7
        prev = uidx_s[jnp.maximum(kc - 1, 0)] >> 7
        return (kc == 0) | (blk != prev)

    def fire_user(k, slot):
        u = uidx_s[jnp.minimum(k, BPW - 1)]
        pltpu.async_copy(
            utT_hbm.at[:, pl.ds((u >> 7) * BLK, BLK)],
            ublk_v.at[slot], usem)

    def fire_item(k, p):
        iv = iidx_s[k]
        pltpu.async_copy(
            itT_hbm.at[:, pl.ds((iv >> 7) * BLK, BLK)],
            iblk_v.at[p], isems[p])

    nf = jnp.int32(0)
    for k0 in range(PF):
        cond = ublock_new(jnp.int32(k0))
        slot = nf & (NBUF - 1)
        pl.when(cond)(lambda k0=k0, slot=slot: fire_user(k0, slot))
        nf = nf + cond.astype(jnp.int32)
        fire_item(k0, k0)

    def ring_body(t, carry):
        nf, cus = carry
        for p in range(NBUF):
            k = t * NBUF + p
            cond_f = (k + PF < BPW) & ublock_new(k + PF)
            fslot = nf & (NBUF - 1)
            pl.when(cond_f)(
                lambda k=k, fslot=fslot: fire_user(k + PF, fslot))
            nf = nf + cond_f.astype(jnp.int32)
            pl.when(k + PF < BPW)(
                lambda k=k, p=p: fire_item(k + PF, (p + PF) % NBUF))

            cond_w = ublock_new(k)

            @pl.when(cond_w)
            def _():
                pltpu.make_async_copy(
                    utT_hbm.at[:, pl.ds(0, BLK)], ublk_v.at[0], usem).wait()

            cus = cus + cond_w.astype(jnp.int32)
            uslot = jnp.full((16,), (cus - 1) & (NBUF - 1), jnp.int32)
            pltpu.make_async_copy(
                itT_hbm.at[:, pl.ds(0, BLK)], iblk_v.at[p], isems[p]).wait()

            cu = jnp.full((16,), uidx_s[k] & 127, jnp.int32)
            ci = jnp.full((16,), iidx_s[k] & 127, jnp.int32)
            kk = jnp.full((16,), k, jnp.int32)
            for s in range(4):
                fvec = s * 16 + lane
                uval = plsc.load_gather(ublk_v, [uslot, fvec, cu])
                ival = plsc.load_gather(iblk_v.at[p], [fvec, ci])
                plsc.store_scatter(prodT_v, [fvec, kk], uval * ival)
        return nf, cus

    lax.fori_loop(0, BPW // NBUF, ring_body, (nf, jnp.int32(0)))

    def group_body(g, carry):
        acc = jnp.zeros((16,), jnp.float32)
        for f in range(F):
            acc = acc + prodT_v[f, pl.ds(g * 16, 16)]
        out_v[pl.ds(g * 16, 16)] = 1.0 / (1.0 + jnp.exp(-acc))
        return carry

    lax.fori_loop(0, BPW // 16, group_body, 0)

    pltpu.sync_copy(out_v, out_hbm.at[wid])


def kernel(user, item, user_table, item_table):
    perm = jnp.argsort(user)
    inv = jnp.argsort(perm)
    us = user[perm].astype(jnp.int32).reshape(NW, BPW)
    its = item[perm].astype(jnp.int32).reshape(NW, BPW)
    outs = _cf_kernel(us, its, user_table.T, item_table.T)
    return outs.reshape(B)[inv]
